# baseline (speedup 1.0000x reference)
import functools

import jax
import jax.numpy as jnp
from jax import lax
from jax.experimental import pallas as pl
from jax.experimental.pallas import tpu as pltpu, tpu_sc as plsc

N = 10000
E = 320000
D = 128
H = 128
C = 2

NC = 2
NS = 16
LANES = 128
CPW = 80
EPW = CPW * LANES
E_PAD = NC * NS * EPW
ROWS_PER_TILE = 640
N_PAD = NS * ROWS_PER_TILE
CHUNKS = E_PAD // LANES

K0 = 608
K1 = CHUNKS - K0
P0 = K0 // NS
P1 = K1 // NS

_mesh = plsc.VectorSubcoreMesh(core_axis_name="c", subcore_axis_name="s")


def _fill2d(ref, nrows, ncols, val):
    def row(i, _):
        for j in range(ncols // 16):
            ref[i, pl.ds(j * 16, 16)] = jnp.full((16,), val, jnp.float32)
        return 0
    lax.fori_loop(0, nrows, row, 0)


@functools.partial(
    pl.kernel,
    out_type=jax.ShapeDtypeStruct((NC, N_PAD, 16), jnp.float32),
    mesh=_mesh,
    scratch_types=[
        pltpu.VMEM((CPW, LANES), jnp.int32),
        pltpu.VMEM((LANES, 16), jnp.float32),
        pltpu.VMEM((LANES, 16), jnp.float32),
        pltpu.VMEM_SHARED((N_PAD, 16), jnp.float32),
    ],
)
def _deg_kernel(dst_hbm, out_hbm, dst_v, ones_v, z_v, acc):
    c = lax.axis_index("c")
    s = lax.axis_index("s")
    pltpu.sync_copy(dst_hbm.at[c, s], dst_v)
    _fill2d(ones_v, LANES, 16, 1.0)
    _fill2d(z_v, LANES, 16, 0.0)
    for k in range(ROWS_PER_TILE // LANES):
        pltpu.sync_copy(z_v, acc.at[pl.ds(s * ROWS_PER_TILE + k * LANES, LANES)])
    plsc.subcore_barrier()

    def chunk(j, _):
        pltpu.sync_copy(ones_v, acc.at[dst_v.at[j]], add=True)
        return 0
    lax.fori_loop(0, CPW, chunk, 0)
    plsc.subcore_barrier()
    pltpu.sync_copy(acc.at[pl.ds(s * ROWS_PER_TILE, ROWS_PER_TILE)],
                    out_hbm.at[c, pl.ds(s * ROWS_PER_TILE, ROWS_PER_TILE)])


@functools.partial(
    pl.kernel,
    out_type=jax.ShapeDtypeStruct((NC, N_PAD, D), jnp.float32),
    mesh=_mesh,
    scratch_types=[
        pltpu.VMEM((2, 2, LANES), jnp.int32),
        pltpu.VMEM((2, LANES, D), jnp.float32),
        pltpu.SemaphoreType.DMA,
        pltpu.SemaphoreType.DMA,
        pltpu.SemaphoreType.DMA,
        pltpu.SemaphoreType.DMA,
        pltpu.VMEM_SHARED((N_PAD, D), jnp.float32),
    ],
)
def _agg_kernel(u_hbm, e_hbm, out_hbm, eb, rows, si0, si1, sg0, sg1, acc):
    c = lax.axis_index("c")
    s = lax.axis_index("s")
    sem_i = (si0, si1)
    sem_g = (sg0, sg1)
    nch = jnp.where(c == 0, P0, P1)
    base = c * K0 + s * nch
    pltpu.async_copy(e_hbm.at[base], eb.at[0], sem_i[0])
    pltpu.async_copy(e_hbm.at[base + 1], eb.at[1], sem_i[1])
    _fill2d(rows.at[0], LANES, D, 0.0)
    for k in range(ROWS_PER_TILE // LANES):
        pltpu.sync_copy(rows.at[0],
                        acc.at[pl.ds(s * ROWS_PER_TILE + k * LANES, LANES)])
    plsc.subcore_barrier()
    pltpu.make_async_copy(e_hbm.at[base], eb.at[0], sem_i[0]).wait()
    pltpu.async_copy(u_hbm.at[eb.at[0, 0]], rows.at[0], sem_g[0])

    def chunk_pair(i, _):
        for b in (0, 1):
            jj = 2 * i + b
            po = 1 - b
            pltpu.make_async_copy(u_hbm.at[eb.at[b, 0]], rows.at[b],
                                  sem_g[b]).wait()

            @pl.when(jj + 1 < nch)
            def _():
                pltpu.make_async_copy(e_hbm.at[base + jj + 1], eb.at[po],
                                      sem_i[po]).wait()
                pltpu.async_copy(u_hbm.at[eb.at[po, 0]], rows.at[po], sem_g[po])

            pltpu.sync_copy(rows.at[b], acc.at[eb.at[b, 1]], add=True)

            @pl.when(jj + 2 < nch)
            def _():
                pltpu.async_copy(e_hbm.at[base + jj + 2], eb.at[b], sem_i[b])
        return 0
    lax.fori_loop(0, nch // 2, chunk_pair, 0)
    plsc.subcore_barrier()
    pltpu.sync_copy(acc.at[pl.ds(s * ROWS_PER_TILE, ROWS_PER_TILE)],
                    out_hbm.at[c, pl.ds(s * ROWS_PER_TILE, ROWS_PER_TILE)])


BN = 1000
_GRID = N // BN


def _row_spec():
    return pl.BlockSpec((BN, 128), lambda i: (i, 0))


def _acc_spec():
    return pl.BlockSpec((2, BN, 128), lambda i: (0, i, 0))


def _cat(ref):
    return ref[0] + ref[1]


def _tc1_body(x_ref, w_ref, d0_ref, d1_ref, u_ref, dinv_ref):
    deg = 1.0 + d0_ref[:, 0:1] + d1_ref[:, 0:1]
    dinv = lax.rsqrt(deg)
    dinv_b = jnp.broadcast_to(dinv, (BN, 128))
    p = jnp.dot(x_ref[...], w_ref[...], preferred_element_type=jnp.float32)
    u_ref[...] = dinv_b * p
    dinv_ref[...] = dinv_b


def _tc1(x, w1, d0, d1):
    return pl.pallas_call(
        _tc1_body,
        grid=(_GRID,),
        in_specs=[
            _row_spec(),
            pl.BlockSpec((128, 128), lambda i: (0, 0)),
            pl.BlockSpec((BN, 16), lambda i: (i, 0)),
            pl.BlockSpec((BN, 16), lambda i: (i, 0)),
        ],
        out_specs=[_row_spec(), _row_spec()],
        out_shape=[
            jax.ShapeDtypeStruct((N, 128), jnp.float32),
            jax.ShapeDtypeStruct((N, 128), jnp.float32),
        ],
    )(x, w1, d0, d1)


def _tc_mid_body(a_ref, u_ref, dinv_ref, b_ref, w_ref, out_ref):
    h = dinv_ref[...] * (_cat(a_ref) + u_ref[...]) + b_ref[...]
    h = jnp.maximum(h, 0.0)
    out_ref[...] = dinv_ref[...] * jnp.dot(
        h, w_ref[...], preferred_element_type=jnp.float32)


def _tc_mid(a, u, dinv_b, b, w):
    return pl.pallas_call(
        _tc_mid_body,
        grid=(_GRID,),
        in_specs=[
            _acc_spec(), _row_spec(), _row_spec(),
            pl.BlockSpec((1, 128), lambda i: (0, 0)),
            pl.BlockSpec((128, 128), lambda i: (0, 0)),
        ],
        out_specs=_row_spec(),
        out_shape=jax.ShapeDtypeStruct((N, 128), jnp.float32),
    )(a, u, dinv_b, b, w)


def _tc3_body(a_ref, u_ref, dinv_ref, b_ref, out_ref):
    h = dinv_ref[...] * (_cat(a_ref) + u_ref[...]) + b_ref[...]
    out_ref[...] = dinv_ref[...] * jnp.maximum(h, 0.0)


def _tc3(a, u, dinv_b, b):
    return pl.pallas_call(
        _tc3_body,
        grid=(_GRID,),
        in_specs=[
            _acc_spec(), _row_spec(), _row_spec(),
            pl.BlockSpec((1, 128), lambda i: (0, 0)),
        ],
        out_specs=_row_spec(),
        out_shape=jax.ShapeDtypeStruct((N, 128), jnp.float32),
    )(a, u, dinv_b, b)


def _tc4_body(a_ref, v_ref, dinv_ref, w_ref, b_ref, out_ref):
    z = dinv_ref[...] * (_cat(a_ref) + v_ref[...])
    out_ref[...] = jnp.dot(z, w_ref[...],
                           preferred_element_type=jnp.float32) + b_ref[...]


def _tc4(a, v, dinv_b, w3p, b3p):
    return pl.pallas_call(
        _tc4_body,
        grid=(_GRID,),
        in_specs=[
            _acc_spec(), _row_spec(), _row_spec(),
            pl.BlockSpec((128, 128), lambda i: (0, 0)),
            pl.BlockSpec((1, 128), lambda i: (0, 0)),
        ],
        out_specs=_row_spec(),
        out_shape=jax.ShapeDtypeStruct((N, 128), jnp.float32),
    )(a, v, dinv_b, w3p, b3p)


def kernel(x, edge_index, W1, b1, W2, b2, W3, b3):
    src = edge_index[0]
    dst = edge_index[1]
    pad = E_PAD - E
    src_p = jnp.concatenate([src, jnp.zeros((pad,), jnp.int32)])
    dst_p = jnp.concatenate([dst, jnp.full((pad,), N, jnp.int32)])
    dst3 = dst_p.reshape(NC, NS, CPW, LANES)
    e3 = jnp.stack([src_p.reshape(CHUNKS, LANES),
                    dst_p.reshape(CHUNKS, LANES)], axis=1)

    degp = _deg_kernel(dst3)
    d0 = degp[0, :N]
    d1 = degp[1, :N]

    u1, dinv_b = _tc1(x, W1, d0, d1)
    a1 = _agg_kernel(u1, e3)
    u2 = _tc_mid(a1, u1, dinv_b, b1.reshape(1, 128), W2)
    a2 = _agg_kernel(u2, e3)
    v3 = _tc3(a2, u2, dinv_b, b2.reshape(1, 128))
    a3 = _agg_kernel(v3, e3)
    w3p = jnp.pad(W3, ((0, 0), (0, 128 - C)))
    b3p = jnp.pad(b3, (0, 128 - C)).reshape(1, 128)
    outp = _tc4(a3, v3, dinv_b, w3p, b3p)
    return outp[:, :C]


# speedup vs baseline: 1.2006x; 1.2006x over previous
import functools

import jax
import jax.numpy as jnp
from jax import lax
from jax.experimental import pallas as pl
from jax.experimental.pallas import tpu as pltpu, tpu_sc as plsc

N = 10000
E = 320000
D = 128
H = 128
C = 2

NC = 2
NS = 16
LANES = 128
CPW = 80
EPW = CPW * LANES
E_PAD = NC * NS * EPW
ROWS_PER_TILE = 640
N_PAD = NS * ROWS_PER_TILE
CHUNKS = E_PAD // LANES

K0 = 1952
K1 = CHUNKS - K0
P0 = K0 // NS
P1 = K1 // NS

_mesh = plsc.VectorSubcoreMesh(core_axis_name="c", subcore_axis_name="s")


def _fill2d(ref, nrows, ncols, val):
    def row(i, _):
        for j in range(ncols // 16):
            ref[i, pl.ds(j * 16, 16)] = jnp.full((16,), val, jnp.float32)
        return 0
    lax.fori_loop(0, nrows, row, 0)


@functools.partial(
    pl.kernel,
    out_type=jax.ShapeDtypeStruct((NC, N_PAD, 16), jnp.float32),
    mesh=_mesh,
    scratch_types=[
        pltpu.VMEM((CPW, LANES), jnp.int32),
        pltpu.VMEM((LANES, 16), jnp.float32),
        pltpu.VMEM((LANES, 16), jnp.float32),
        pltpu.VMEM_SHARED((N_PAD, 16), jnp.float32),
    ],
)
def _deg_kernel(dst_hbm, out_hbm, dst_v, ones_v, z_v, acc):
    c = lax.axis_index("c")
    s = lax.axis_index("s")
    pltpu.sync_copy(dst_hbm.at[c, s], dst_v)
    _fill2d(ones_v, LANES, 16, 1.0)
    _fill2d(z_v, LANES, 16, 0.0)
    for k in range(ROWS_PER_TILE // LANES):
        pltpu.sync_copy(z_v, acc.at[pl.ds(s * ROWS_PER_TILE + k * LANES, LANES)])
    plsc.subcore_barrier()

    def chunk(j, _):
        pltpu.sync_copy(ones_v, acc.at[dst_v.at[j]], add=True)
        return 0
    lax.fori_loop(0, CPW, chunk, 0)
    plsc.subcore_barrier()
    pltpu.sync_copy(acc.at[pl.ds(s * ROWS_PER_TILE, ROWS_PER_TILE)],
                    out_hbm.at[c, pl.ds(s * ROWS_PER_TILE, ROWS_PER_TILE)])


@functools.partial(
    pl.kernel,
    out_type=jax.ShapeDtypeStruct((NC, N_PAD, D), jnp.float32),
    mesh=_mesh,
    scratch_types=[
        pltpu.VMEM((2, 2, LANES), jnp.int32),
        pltpu.VMEM((2, LANES, D), jnp.float32),
        pltpu.SemaphoreType.DMA,
        pltpu.SemaphoreType.DMA,
        pltpu.SemaphoreType.DMA,
        pltpu.SemaphoreType.DMA,
        pltpu.VMEM_SHARED((N_PAD, D), jnp.float32),
    ],
)
def _agg_kernel(u_hbm, e_hbm, out_hbm, eb, rows, si0, si1, sg0, sg1, acc):
    c = lax.axis_index("c")
    s = lax.axis_index("s")
    sem_i = (si0, si1)
    sem_g = (sg0, sg1)
    nch = jnp.where(c == 0, P0, P1)
    base = c * K0 + s * nch
    pltpu.async_copy(e_hbm.at[base], eb.at[0], sem_i[0])
    pltpu.async_copy(e_hbm.at[base + 1], eb.at[1], sem_i[1])
    _fill2d(rows.at[0], LANES, D, 0.0)
    for k in range(ROWS_PER_TILE // LANES):
        pltpu.sync_copy(rows.at[0],
                        acc.at[pl.ds(s * ROWS_PER_TILE + k * LANES, LANES)])
    plsc.subcore_barrier()
    pltpu.make_async_copy(e_hbm.at[base], eb.at[0], sem_i[0]).wait()
    pltpu.async_copy(u_hbm.at[eb.at[0, 0]], rows.at[0], sem_g[0])

    def chunk_pair(i, _):
        for b in (0, 1):
            jj = 2 * i + b
            po = 1 - b
            pltpu.make_async_copy(u_hbm.at[eb.at[b, 0]], rows.at[b],
                                  sem_g[b]).wait()

            @pl.when(jj + 1 < nch)
            def _():
                pltpu.make_async_copy(e_hbm.at[base + jj + 1], eb.at[po],
                                      sem_i[po]).wait()
                pltpu.async_copy(u_hbm.at[eb.at[po, 0]], rows.at[po], sem_g[po])

            pltpu.sync_copy(rows.at[b], acc.at[eb.at[b, 1]], add=True)

            @pl.when(jj + 2 < nch)
            def _():
                pltpu.async_copy(e_hbm.at[base + jj + 2], eb.at[b], sem_i[b])
        return 0
    lax.fori_loop(0, nch // 2, chunk_pair, 0)
    plsc.subcore_barrier()
    pltpu.sync_copy(acc.at[pl.ds(s * ROWS_PER_TILE, ROWS_PER_TILE)],
                    out_hbm.at[c, pl.ds(s * ROWS_PER_TILE, ROWS_PER_TILE)])


BN = 1000
_GRID = N // BN


def _row_spec():
    return pl.BlockSpec((BN, 128), lambda i: (i, 0))


def _acc_spec():
    return pl.BlockSpec((2, BN, 128), lambda i: (0, i, 0))


def _cat(ref):
    return ref[0] + ref[1]


def _tc1_body(x_ref, w_ref, d0_ref, d1_ref, u_ref, dinv_ref):
    deg = 1.0 + d0_ref[:, 0:1] + d1_ref[:, 0:1]
    dinv = lax.rsqrt(deg)
    dinv_b = jnp.broadcast_to(dinv, (BN, 128))
    p = jnp.dot(x_ref[...], w_ref[...], preferred_element_type=jnp.float32)
    u_ref[...] = dinv_b * p
    dinv_ref[...] = dinv_b


def _tc1(x, w1, d0, d1):
    return pl.pallas_call(
        _tc1_body,
        grid=(_GRID,),
        in_specs=[
            _row_spec(),
            pl.BlockSpec((128, 128), lambda i: (0, 0)),
            pl.BlockSpec((BN, 16), lambda i: (i, 0)),
            pl.BlockSpec((BN, 16), lambda i: (i, 0)),
        ],
        out_specs=[_row_spec(), _row_spec()],
        out_shape=[
            jax.ShapeDtypeStruct((N, 128), jnp.float32),
            jax.ShapeDtypeStruct((N, 128), jnp.float32),
        ],
    )(x, w1, d0, d1)


def _tc_mid_body(a_ref, u_ref, dinv_ref, b_ref, w_ref, out_ref):
    h = dinv_ref[...] * (_cat(a_ref) + u_ref[...]) + b_ref[...]
    h = jnp.maximum(h, 0.0)
    out_ref[...] = dinv_ref[...] * jnp.dot(
        h, w_ref[...], preferred_element_type=jnp.float32)


def _tc_mid(a, u, dinv_b, b, w):
    return pl.pallas_call(
        _tc_mid_body,
        grid=(_GRID,),
        in_specs=[
            _acc_spec(), _row_spec(), _row_spec(),
            pl.BlockSpec((1, 128), lambda i: (0, 0)),
            pl.BlockSpec((128, 128), lambda i: (0, 0)),
        ],
        out_specs=_row_spec(),
        out_shape=jax.ShapeDtypeStruct((N, 128), jnp.float32),
    )(a, u, dinv_b, b, w)


def _tc3_body(a_ref, u_ref, dinv_ref, b_ref, out_ref):
    h = dinv_ref[...] * (_cat(a_ref) + u_ref[...]) + b_ref[...]
    out_ref[...] = dinv_ref[...] * jnp.maximum(h, 0.0)


def _tc3(a, u, dinv_b, b):
    return pl.pallas_call(
        _tc3_body,
        grid=(_GRID,),
        in_specs=[
            _acc_spec(), _row_spec(), _row_spec(),
            pl.BlockSpec((1, 128), lambda i: (0, 0)),
        ],
        out_specs=_row_spec(),
        out_shape=jax.ShapeDtypeStruct((N, 128), jnp.float32),
    )(a, u, dinv_b, b)


def _tc4_body(a_ref, v_ref, dinv_ref, w_ref, b_ref, out_ref):
    z = dinv_ref[...] * (_cat(a_ref) + v_ref[...])
    out_ref[...] = jnp.dot(z, w_ref[...],
                           preferred_element_type=jnp.float32) + b_ref[...]


def _tc4(a, v, dinv_b, w3p, b3p):
    return pl.pallas_call(
        _tc4_body,
        grid=(_GRID,),
        in_specs=[
            _acc_spec(), _row_spec(), _row_spec(),
            pl.BlockSpec((128, 128), lambda i: (0, 0)),
            pl.BlockSpec((1, 128), lambda i: (0, 0)),
        ],
        out_specs=_row_spec(),
        out_shape=jax.ShapeDtypeStruct((N, 128), jnp.float32),
    )(a, v, dinv_b, w3p, b3p)


def kernel(x, edge_index, W1, b1, W2, b2, W3, b3):
    src = edge_index[0]
    dst = edge_index[1]
    pad = E_PAD - E
    src_p = jnp.concatenate([src, jnp.zeros((pad,), jnp.int32)])
    dst_p = jnp.concatenate([dst, jnp.full((pad,), N, jnp.int32)])
    dst3 = dst_p.reshape(NC, NS, CPW, LANES)
    e3 = jnp.stack([src_p.reshape(CHUNKS, LANES),
                    dst_p.reshape(CHUNKS, LANES)], axis=1)

    degp = _deg_kernel(dst3)
    d0 = degp[0, :N]
    d1 = degp[1, :N]

    u1, dinv_b = _tc1(x, W1, d0, d1)
    a1 = _agg_kernel(u1, e3)
    u2 = _tc_mid(a1, u1, dinv_b, b1.reshape(1, 128), W2)
    a2 = _agg_kernel(u2, e3)
    v3 = _tc3(a2, u2, dinv_b, b2.reshape(1, 128))
    a3 = _agg_kernel(v3, e3)
    w3p = jnp.pad(W3, ((0, 0), (0, 128 - C)))
    b3p = jnp.pad(b3, (0, 128 - C)).reshape(1, 128)
    outp = _tc4(a3, v3, dinv_b, w3p, b3p)
    return outp[:, :C]
